# baseline (speedup 1.0000x reference)
import functools

import jax
import jax.numpy as jnp
from jax import lax
from jax.experimental import pallas as pl
from jax.experimental.pallas import tpu as pltpu
from jax.experimental.pallas import tpu_sc as plsc

_NC = 2
_NS = 16
_NW = _NC * _NS
_LANES = 16
_CHUNK = 128
_PACK = 4


def _scores_kernel(B, D, users_hbm, items_hbm, ut_hbm, it_hbm, out_hbm,
                   uraw, iraw, uvi, ivi, ubuf, ibuf, oscr, idx_sem,
                   usem0, usem1, isem0, isem1):
    bpw = B // _NW
    nchunk = bpw // _CHUNK
    wid = lax.axis_index("s") * _NC + lax.axis_index("c")
    base = wid * bpw
    usems = (usem0, usem1)
    isems = (isem0, isem1)

    idx_copies = []
    for j in range(nchunk):
        idx_copies.append(pltpu.async_copy(
            users_hbm.at[pl.ds(base + j * _CHUNK, _CHUNK)], uraw.at[j],
            idx_sem))
        idx_copies.append(pltpu.async_copy(
            items_hbm.at[pl.ds(base + j * _CHUNK, _CHUNK)], iraw.at[j],
            idx_sem))
    for c in idx_copies:
        c.wait()

    for j in range(nchunk):
        for v in range(_CHUNK // _LANES):
            sl = pl.ds(v * _LANES, _LANES)
            uvi[j, sl] = lax.shift_right_logical(uraw[j, sl], 2)
            ivi[j, sl] = lax.shift_right_logical(iraw[j, sl], 2)

    def fire(j):
        s = j % 2
        cu = pltpu.async_copy(ut_hbm.at[uvi.at[j]], ubuf.at[s], usems[s])
        ci = pltpu.async_copy(it_hbm.at[ivi.at[j]], ibuf.at[s], isems[s])
        return cu, ci

    lanes = lax.iota(jnp.int32, _LANES)
    inflight = {}
    inflight[0] = fire(0)
    if nchunk > 1:
        inflight[1] = fire(1)

    for j in range(nchunk):
        s = j % 2
        cu, ci = inflight.pop(j)
        cu.wait()
        ci.wait()

        def group_body(g, _):
            sl = pl.ds(g * _LANES, _LANES)
            rows = g * _LANES + lanes
            ur = uraw[j, sl]
            ir = iraw[j, sl]
            ucol = (ur & (_PACK - 1)) * D
            icol = (ir & (_PACK - 1)) * D
            acc = jnp.zeros((_LANES,), jnp.float32)
            for d in range(D):
                uv = plsc.load_gather(ubuf, [jnp.full((_LANES,), s, jnp.int32),
                                             rows, ucol + d])
                iv = plsc.load_gather(ibuf, [jnp.full((_LANES,), s, jnp.int32),
                                             rows, icol + d])
                acc = acc + uv * iv
            sig = 1.0 / (1.0 + jnp.exp(-acc))
            oscr[pl.ds(j * _CHUNK + g * _LANES, _LANES)] = sig
            return 0

        lax.fori_loop(0, _CHUNK // _LANES, group_body, 0)
        if j + 2 < nchunk:
            inflight[j + 2] = fire(j + 2)

    pltpu.sync_copy(oscr, out_hbm.at[pl.ds(base, bpw)])


def kernel(users, items, user_table, item_table):
    B = users.shape[0]
    V, D = user_table.shape
    bpw = B // _NW
    nchunk = bpw // _CHUNK
    W = _PACK * D
    ut_view = user_table.reshape(V // _PACK, W)
    it_view = item_table.reshape(V // _PACK, W)
    mesh = plsc.VectorSubcoreMesh(core_axis_name="c", subcore_axis_name="s")

    run = functools.partial(
        pl.kernel,
        mesh=mesh,
        compiler_params=pltpu.CompilerParams(
            needs_layout_passes=False, use_tc_tiling_on_sc=True),
        out_type=jax.ShapeDtypeStruct((B,), jnp.float32),
        scratch_types=[
            pltpu.VMEM((nchunk, _CHUNK), jnp.int32),
            pltpu.VMEM((nchunk, _CHUNK), jnp.int32),
            pltpu.VMEM((nchunk, _CHUNK), jnp.int32),
            pltpu.VMEM((nchunk, _CHUNK), jnp.int32),
            pltpu.VMEM((2, _CHUNK, W), jnp.float32),
            pltpu.VMEM((2, _CHUNK, W), jnp.float32),
            pltpu.VMEM((bpw,), jnp.float32),
            pltpu.SemaphoreType.DMA,
            pltpu.SemaphoreType.DMA,
            pltpu.SemaphoreType.DMA,
            pltpu.SemaphoreType.DMA,
            pltpu.SemaphoreType.DMA,
        ],
    )(functools.partial(_scores_kernel, B, D))
    return run(users, items, ut_view, it_view)
